# baseline (speedup 1.0000x reference)
import jax
import jax.numpy as jnp
from jax.experimental import pallas as pl

_TN = 2048


def _nnd_tile_kernel(x_ref, y_ref, o1_ref, o2_ref):
    x = x_ref[0]
    y = y_ref[0]

    xk = [x[:, k:k + 1] for k in range(3)]
    yk = [y[:, k:k + 1] for k in range(3)]

    x2 = (xk[0] * xk[0] + xk[1] * xk[1]) + xk[2] * xk[2]
    y2c = (yk[0] * yk[0] + yk[1] * yk[1]) + yk[2] * yk[2]
    y2 = y2c.reshape(1, y.shape[0])

    xb2 = x.astype(jnp.bfloat16) * jnp.bfloat16(-2.0)
    yb = y.astype(jnp.bfloat16)
    xy2 = jax.lax.dot_general(
        xb2, yb,
        (((1,), (1,)), ((), ())),
        preferred_element_type=jnp.float32,
    )

    d = (x2 + y2) + xy2

    tn = d.shape[0]
    o1_ref[...] = jnp.min(d, axis=1, keepdims=True).reshape(1, tn, 1)
    o2_ref[...] = jnp.min(d, axis=0, keepdims=True)[None]


def kernel(input1, input2):
    b, n, _ = input1.shape
    m = input2.shape[1]

    grid = (b, n // _TN)
    out1, out2 = pl.pallas_call(
        _nnd_tile_kernel,
        grid=grid,
        in_specs=[
            pl.BlockSpec((1, _TN, 3), lambda bi, ni: (bi, ni, 0)),
            pl.BlockSpec((1, m, 3), lambda bi, ni: (bi, 0, 0)),
        ],
        out_specs=[
            pl.BlockSpec((1, _TN, 1), lambda bi, ni: (bi, ni, 0)),
            pl.BlockSpec((1, 1, m), lambda bi, ni: (bi, 0, 0)),
        ],
        out_shape=[
            jax.ShapeDtypeStruct((b, n, 1), jnp.float32),
            jax.ShapeDtypeStruct((b, 1, m), jnp.float32),
        ],
    )(input1, input2)

    return out1[:, :, 0], out2[:, 0, :]
